# baseline (speedup 1.0000x reference)
import numpy as np
import jax
import jax.numpy as jnp
from jax.experimental import pallas as pl
from jax.experimental.pallas import tpu as pltpu

B = 256
N_CLIN = 38
N_PIX = 36
FV = 128
SPB = 32
NCHUNK = B // SPB

RC = SPB * N_CLIN
RI = SPB * N_PIX

_rows_c = np.arange(RC) // N_CLIN
_rows_i = np.arange(RI) // N_PIX
_PC = (np.arange(SPB)[:, None] == _rows_c[None, :]).astype(np.float32)
_PI = (np.arange(SPB)[:, None] == _rows_i[None, :]).astype(np.float32)
_QC = _PC.T.copy()
_QI = _PI.T.copy()
_TC = (np.arange(RC)[:, None] % N_CLIN == np.arange(N_CLIN)[None, :]).astype(np.float32)


def _pipe_kernel(clin_hbm, img_hbm, wg_ref, w39_ref, bias_ref,
                 pc_ref, pi_ref, qc_ref, qi_ref, tc_ref, out_ref,
                 cbuf, ibuf, csem, isem):
    wg = wg_ref[...]
    w39 = w39_ref[...]
    bias = bias_ref[0, 0]

    dot = lambda a, b: jnp.dot(a, b, preferred_element_type=jnp.float32)

    def start_copy(c, slot):
        pltpu.make_async_copy(
            clin_hbm.at[pl.ds(c * RC, RC), :], cbuf.at[slot], csem.at[slot]
        ).start()
        pltpu.make_async_copy(
            img_hbm.at[pl.ds(c * RI, RI), :], ibuf.at[slot], isem.at[slot]
        ).start()

    def wait_copy(c, slot):
        pltpu.make_async_copy(
            clin_hbm.at[pl.ds(c * RC, RC), :], cbuf.at[slot], csem.at[slot]
        ).wait()
        pltpu.make_async_copy(
            img_hbm.at[pl.ds(c * RI, RI), :], ibuf.at[slot], isem.at[slot]
        ).wait()

    start_copy(0, 0)

    wtile_c = dot(tc_ref[...], w39[:N_CLIN, :])
    wtile_i = jnp.broadcast_to(w39[N_CLIN:, :] * (1.0 / N_PIX), (RI, FV))

    def body(c, carry):
        slot = jax.lax.rem(c, 2)
        nslot = jax.lax.rem(c + 1, 2)

        @pl.when(c + 1 < NCHUNK)
        def _():
            start_copy(c + 1, nslot)

        wait_copy(c, slot)
        clin = cbuf[slot]
        img = ibuf[slot]

        s_clin = dot(pc_ref[...], clin)
        s_pix = dot(pi_ref[...], img)
        agg_c = clin + dot(qc_ref[...], s_pix)
        agg_i = img + dot(qi_ref[...], s_clin)
        h_c = jnp.maximum(dot(agg_c, wg), 0.0)
        h_i = jnp.maximum(dot(agg_i, wg), 0.0)
        z = dot(pc_ref[...], h_c * wtile_c) + dot(pi_ref[...], h_i * wtile_i)
        out_ref[pl.ds(c * SPB, SPB), :] = (
            jnp.sum(z, axis=1, keepdims=True) + bias
        )
        return carry

    jax.lax.fori_loop(0, NCHUNK, body, 0)


def kernel(clinical_embeddings, image_embeddings, edge_index, W_g, W_out, b_out):
    del edge_index
    clin = clinical_embeddings.reshape(B * N_CLIN, FV)
    img = image_embeddings.reshape(B * N_PIX, FV)
    w39 = W_out.reshape(N_CLIN + 1, FV)
    bias = b_out.reshape(1, 1)
    vspec = lambda: pl.BlockSpec(memory_space=pltpu.VMEM)
    return pl.pallas_call(
        _pipe_kernel,
        in_specs=[
            pl.BlockSpec(memory_space=pl.ANY),
            pl.BlockSpec(memory_space=pl.ANY),
            vspec(), vspec(), vspec(), vspec(), vspec(), vspec(), vspec(), vspec(),
        ],
        out_specs=pl.BlockSpec(memory_space=pltpu.VMEM),
        out_shape=jax.ShapeDtypeStruct((B, 1), jnp.float32),
        scratch_shapes=[
            pltpu.VMEM((2, RC, FV), jnp.float32),
            pltpu.VMEM((2, RI, FV), jnp.float32),
            pltpu.SemaphoreType.DMA((2,)),
            pltpu.SemaphoreType.DMA((2,)),
        ],
    )(clin, img, W_g, w39, bias,
      jnp.asarray(_PC), jnp.asarray(_PI), jnp.asarray(_QC), jnp.asarray(_QI),
      jnp.asarray(_TC))
